# Initial kernel scaffold; baseline (speedup 1.0000x reference)
import jax
import jax.numpy as jnp
from jax.experimental import pallas as pl


def kernel(input_ids, table):
    raise NotImplementedError("write your pallas kernel here")


# speedup vs baseline: 3.9566x; 3.9566x over previous
import functools

import jax
import jax.numpy as jnp
from jax import lax
from jax.experimental import pallas as pl
from jax.experimental.pallas import tpu as pltpu
from jax.experimental.pallas import tpu_sc as plsc

_IDXW = 128
_G = 4


def _emb_call(n_rows_pw, n_chunks, V, D, idx2, table):
    mesh = plsc.VectorSubcoreMesh(core_axis_name="c", subcore_axis_name="s")
    NC = 2
    CB = _G * _IDXW

    @functools.partial(
        pl.kernel,
        out_type=jax.ShapeDtypeStruct((idx2.shape[0] * _IDXW, D), jnp.float32),
        mesh=mesh,
        scratch_types=[
            pltpu.VMEM((_G, _IDXW), jnp.int32),
            pltpu.VMEM((CB, D), jnp.float32),
            pltpu.SemaphoreType.DMA,
        ],
        compiler_params=pltpu.CompilerParams(use_tc_tiling_on_sc=False),
    )
    def emb(idx_hbm, table_hbm, out_hbm, idx_v, rows_v, gsem):
        wid = lax.axis_index("s") * NC + lax.axis_index("c")
        row0 = wid * n_rows_pw

        def chunk(c, _):
            r = row0 + c * _G
            pltpu.sync_copy(idx_hbm.at[pl.ds(r, _G)], idx_v)
            copies = []
            for j in range(_G):
                copies.append(
                    pltpu.async_copy(
                        table_hbm.at[idx_v.at[j]],
                        rows_v.at[pl.ds(j * _IDXW, _IDXW)],
                        gsem,
                    )
                )
            for cp in copies:
                cp.wait()
            pltpu.sync_copy(rows_v, out_hbm.at[pl.ds(r * _IDXW, CB)])
            return 0

        lax.fori_loop(0, n_chunks, chunk, 0)

    return emb(idx2, table)


def kernel(input_ids, table):
    B0, S = input_ids.shape
    V, D = table.shape
    B = B0 * S
    NW = 32
    assert B % (NW * _G * _IDXW) == 0
    n_rows_pw = B // (NW * _IDXW)
    n_chunks = n_rows_pw // _G
    idx2 = input_ids.reshape(B // _IDXW, _IDXW)
    out = _emb_call(n_rows_pw, n_chunks, V, D, idx2, table)
    return out.reshape(B0, S, D)


# speedup vs baseline: 4.1806x; 1.0566x over previous
import functools

import jax
import jax.numpy as jnp
from jax import lax
from jax.experimental import pallas as pl
from jax.experimental.pallas import tpu as pltpu
from jax.experimental.pallas import tpu_sc as plsc

_IDXW = 128
_G = 4


def _emb_call(n_rows_pw, n_chunks, D, idx2, table):
    mesh = plsc.VectorSubcoreMesh(core_axis_name="c", subcore_axis_name="s")
    NC = 2
    CB = _G * _IDXW
    n2 = n_chunks // 2

    @functools.partial(
        pl.kernel,
        out_type=jax.ShapeDtypeStruct((idx2.shape[0] * _IDXW, D), jnp.float32),
        mesh=mesh,
        scratch_types=[
            pltpu.VMEM((2, _G, _IDXW), jnp.int32),
            pltpu.VMEM((2, CB, D), jnp.float32),
            pltpu.SemaphoreType.DMA,
            pltpu.SemaphoreType.DMA,
            pltpu.SemaphoreType.DMA,
            pltpu.SemaphoreType.DMA,
        ],
        compiler_params=pltpu.CompilerParams(use_tc_tiling_on_sc=False),
    )
    def emb(idx_hbm, table_hbm, out_hbm, idx_v, rows_v, g0, g1, s0, s1):
        wid = lax.axis_index("s") * NC + lax.axis_index("c")
        row0 = wid * n_rows_pw
        gsem = (g0, g1)
        ssem = (s0, s1)

        def fire(cc, buf):
            r = row0 + cc * _G
            pltpu.sync_copy(idx_hbm.at[pl.ds(r, _G)], idx_v.at[buf])
            for j in range(_G):
                pltpu.async_copy(
                    table_hbm.at[idx_v.at[buf, j]],
                    rows_v.at[buf, pl.ds(j * _IDXW, _IDXW)],
                    gsem[buf],
                )

        def wait_gathers(buf):
            pltpu.make_async_copy(
                table_hbm.at[pl.ds(0, CB)], rows_v.at[buf], gsem[buf]
            ).wait()

        def fire_store(cc, buf):
            r = row0 + cc * _G
            pltpu.async_copy(
                rows_v.at[buf], out_hbm.at[pl.ds(r * _IDXW, CB)], ssem[buf]
            )

        def wait_store(buf):
            pltpu.make_async_copy(
                rows_v.at[buf], out_hbm.at[pl.ds(0, CB)], ssem[buf]
            ).wait()

        fire(0, 0)

        def body(i, _):
            @pl.when(i > 0)
            def _():
                wait_store(1)

            fire(2 * i + 1, 1)
            wait_gathers(0)
            fire_store(2 * i, 0)

            @pl.when(i < n2 - 1)
            def _():
                wait_store(0)
                fire(2 * i + 2, 0)

            wait_gathers(1)
            fire_store(2 * i + 1, 1)
            return 0

        lax.fori_loop(0, n2, body, 0)
        wait_store(0)
        wait_store(1)

    return emb(idx2, table)


def kernel(input_ids, table):
    B0, S = input_ids.shape
    V, D = table.shape
    B = B0 * S
    NW = 32
    assert B % (NW * 2 * _G * _IDXW) == 0
    n_rows_pw = B // (NW * _IDXW)
    n_chunks = n_rows_pw // _G
    idx2 = input_ids.reshape(B // _IDXW, _IDXW)
    out = _emb_call(n_rows_pw, n_chunks, D, idx2, table)
    return out.reshape(B0, S, D)


# speedup vs baseline: 4.2048x; 1.0058x over previous
import functools

import jax
import jax.numpy as jnp
from jax import lax
from jax.experimental import pallas as pl
from jax.experimental.pallas import tpu as pltpu
from jax.experimental.pallas import tpu_sc as plsc

_R = 4
_SPLITS = ((0, 128), (128, 72))


def _emb_call(rows_pw, n_chunks, idx, table):
    B0, S = idx.shape
    V, D = table.shape
    mesh = plsc.VectorSubcoreMesh(core_axis_name="c", subcore_axis_name="s")
    NC = 2
    n2 = n_chunks // 2
    chunk_elems = _R * S * D

    @functools.partial(
        pl.kernel,
        out_type=jax.ShapeDtypeStruct((B0, S, D), jnp.float32),
        mesh=mesh,
        scratch_types=[
            pltpu.VMEM((2, _R, S), jnp.int32),
            pltpu.VMEM((2, _R, S, D), jnp.float32),
            pltpu.SemaphoreType.DMA,
            pltpu.SemaphoreType.DMA,
            pltpu.SemaphoreType.DMA,
            pltpu.SemaphoreType.DMA,
        ],
        compiler_params=pltpu.CompilerParams(use_tc_tiling_on_sc=False),
    )
    def emb(idx_hbm, table_hbm, out_hbm, idx_v, rows_v, g0, g1, s0, s1):
        wid = lax.axis_index("s") * NC + lax.axis_index("c")
        b00 = wid * rows_pw
        gsem = (g0, g1)
        ssem = (s0, s1)

        def fire(cc, buf):
            b0 = b00 + cc * _R
            pltpu.sync_copy(idx_hbm.at[pl.ds(b0, _R)], idx_v.at[buf])
            for i in range(_R):
                for (o, w) in _SPLITS:
                    pltpu.async_copy(
                        table_hbm.at[idx_v.at[buf, i, pl.ds(o, w)]],
                        rows_v.at[buf, i, pl.ds(o, w)],
                        gsem[buf],
                    )

        def wait_gathers(buf):
            pltpu.make_async_copy(
                out_hbm.at[pl.ds(0, _R)], rows_v.at[buf], gsem[buf]
            ).wait()

        def fire_store(cc, buf):
            b0 = b00 + cc * _R
            pltpu.async_copy(
                rows_v.at[buf], out_hbm.at[pl.ds(b0, _R)], ssem[buf]
            )

        def wait_store(buf):
            pltpu.make_async_copy(
                rows_v.at[buf], out_hbm.at[pl.ds(0, _R)], ssem[buf]
            ).wait()

        fire(0, 0)

        def body(i, _):
            @pl.when(i > 0)
            def _():
                wait_store(1)

            fire(2 * i + 1, 1)
            wait_gathers(0)
            fire_store(2 * i, 0)

            @pl.when(i < n2 - 1)
            def _():
                wait_store(0)
                fire(2 * i + 2, 0)

            wait_gathers(1)
            fire_store(2 * i + 1, 1)
            return 0

        lax.fori_loop(0, n2, body, 0)
        wait_store(0)
        wait_store(1)

    return emb(idx, table)


def kernel(input_ids, table):
    B0, S = input_ids.shape
    NW = 32
    assert B0 % (NW * 2 * _R) == 0
    rows_pw = B0 // NW
    n_chunks = rows_pw // _R
    return _emb_call(rows_pw, n_chunks, input_ids, table)


# speedup vs baseline: 7.4993x; 1.7835x over previous
import functools

import jax
import jax.numpy as jnp
from jax import lax
from jax.experimental import pallas as pl
from jax.experimental.pallas import tpu as pltpu
from jax.experimental.pallas import tpu_sc as plsc

_R = 2
_IH = 64
_SPLITS = ((0, 128), (128, 72))


def _emb_call(rows_pw, idx, table):
    B0, S = idx.shape
    V, D = table.shape
    mesh = plsc.VectorSubcoreMesh(core_axis_name="c", subcore_axis_name="s")
    NC = 2
    n_halves = rows_pw // _IH
    n_chunks = _IH // _R
    n2 = n_chunks // 2
    DP = 2 * D

    @functools.partial(
        pl.kernel,
        out_type=jax.ShapeDtypeStruct((B0, S, DP), jnp.float32),
        mesh=mesh,
        scratch_types=[
            pltpu.VMEM((_IH, S), jnp.int32),
            pltpu.VMEM((2, _R, S, D), jnp.float32),
            pltpu.SemaphoreType.DMA,
            pltpu.SemaphoreType.DMA,
            pltpu.SemaphoreType.DMA,
            pltpu.SemaphoreType.DMA,
        ],
        compiler_params=pltpu.CompilerParams(use_tc_tiling_on_sc=False),
    )
    def emb(idx_hbm, table_hbm, out_hbm, idx_v, rows_v, g0, g1, s0, s1):
        wid = lax.axis_index("s") * NC + lax.axis_index("c")
        b00 = wid * rows_pw
        gsem = (g0, g1)
        ssem = (s0, s1)

        def fire(cc, buf):
            for i in range(_R):
                for (o, w) in _SPLITS:
                    pltpu.async_copy(
                        table_hbm.at[idx_v.at[cc * _R + i, pl.ds(o, w)]],
                        rows_v.at[buf, i, pl.ds(o, w)],
                        gsem[buf],
                    )

        def wait_gathers(buf):
            for i in range(_R):
                for (o, w) in _SPLITS:
                    pltpu.make_async_copy(
                        table_hbm.at[idx_v.at[i, pl.ds(o, w)]],
                        rows_v.at[buf, i, pl.ds(o, w)],
                        gsem[buf],
                    ).wait()

        def fire_store(h, cc, buf):
            b0 = b00 + h * _IH + cc * _R
            pltpu.async_copy(
                rows_v.at[buf],
                out_hbm.at[pl.ds(b0, _R), :, pl.ds(0, D)],
                ssem[buf],
            )

        def wait_store(buf):
            pltpu.make_async_copy(
                rows_v.at[buf],
                out_hbm.at[pl.ds(0, _R), :, pl.ds(0, D)],
                ssem[buf],
            ).wait()

        def half(h, _):
            pltpu.sync_copy(idx_hbm.at[pl.ds(b00 + h * _IH, _IH)], idx_v)
            fire(0, 0)

            def body(i, _):
                @pl.when(jnp.logical_or(i > 0, h > 0))
                def _():
                    wait_store(1)

                fire(2 * i + 1, 1)
                wait_gathers(0)
                fire_store(h, 2 * i, 0)

                @pl.when(i < n2 - 1)
                def _():
                    wait_store(0)
                    fire(2 * i + 2, 0)

                wait_gathers(1)
                fire_store(h, 2 * i + 1, 1)
                return 0

            lax.fori_loop(0, n2, body, 0)
            wait_store(0)
            return 0

        lax.fori_loop(0, n_halves, half, 0)
        wait_store(1)

    return emb(idx, table)


def kernel(input_ids, table):
    B0, S = input_ids.shape
    NW = 32
    rows_pw = B0 // NW
    assert rows_pw % _IH == 0 and _IH % (2 * _R) == 0
    out_p = _emb_call(rows_pw, input_ids, table)
    return out_p[..., : table.shape[1]]
